# initial kernel scaffold (unmeasured)
import jax
import jax.numpy as jnp
from jax import lax
from jax.experimental import pallas as pl
from jax.experimental.pallas import tpu as pltpu

B, H, D, BS = 16, 16, 64, 16
NB = 128
P_LOCAL = 128
NEG = -1e30


def kernel(Q, K, V, bt, lens):
    lens2 = lens.reshape(B, 1)

    def body(q_ref, k_ref, v_ref, bt_ref, lens_ref, out_ref,
             acc_comm, stats_comm, send_sems, recv_sems):
        my_x = lax.axis_index("x")
        my_y = lax.axis_index("y")
        my_z = lax.axis_index("z")
        nbr = (1 - my_x, my_y, my_z)

        barrier = pltpu.get_barrier_semaphore()
        pl.semaphore_signal(barrier, inc=1, device_id=nbr,
                            device_id_type=pl.DeviceIdType.MESH)
        pl.semaphore_wait(barrier, 1)

        q = q_ref[:, 0, :, :]
        k = k_ref[...].reshape(P_LOCAL * BS, H, D)
        v = v_ref[...].reshape(P_LOCAL * BS, H, D)
        btv = bt_ref[...]
        lensv = lens_ref[...]

        slot_ok = lax.broadcasted_iota(jnp.int32, (B, NB), 1) < lensv
        page_ids = lax.broadcasted_iota(jnp.int32, (B, NB, P_LOCAL), 2)
        local_bt = btv - my_x * P_LOCAL
        hit = (local_bt[:, :, None] == page_ids) & slot_ok[:, :, None]
        counts = jnp.sum(hit.astype(jnp.float32), axis=1)
        w = jnp.broadcast_to(
            counts[:, :, None], (B, P_LOCAL, BS)
        ).reshape(B, P_LOCAL * BS)
        sel = (w > 0)[:, None, :]

        s = jnp.einsum("bhd,khd->bhk", q, k,
                       preferred_element_type=jnp.float32) * (D ** -0.5)
        m = jnp.max(jnp.where(sel, s, NEG), axis=-1)
        e = w[:, None, :] * jnp.exp(
            jnp.where(sel, s - m[:, :, None], NEG))
        l = jnp.sum(e, axis=-1)
        acc = jnp.einsum("bhk,khd->bhd", e, v,
                         preferred_element_type=jnp.float32)

        acc_comm[0] = acc
        stats_comm[0, 0] = m
        stats_comm[0, 1] = l

        copies = [
            pltpu.make_async_remote_copy(
                src_ref=ref.at[0], dst_ref=ref.at[1],
                send_sem=send_sems.at[i], recv_sem=recv_sems.at[i],
                device_id=nbr, device_id_type=pl.DeviceIdType.MESH,
            )
            for i, ref in enumerate((acc_comm, stats_comm))
        ]
        for c in copies:
            c.start()
        for c in copies:
            c.wait()

        acc2 = acc_comm[1]
        m2 = stats_comm[1, 0]
        l2 = stats_comm[1, 1]
        mm = jnp.maximum(m, m2)
        a1 = jnp.exp(m - mm)
        a2 = jnp.exp(m2 - mm)
        ll = l * a1 + l2 * a2
        o = (acc * a1[:, :, None] + acc2 * a2[:, :, None]) / ll[:, :, None]
        out_ref[...] = o[:, None, :, :]

    return pl.pallas_call(
        body,
        out_shape=jax.ShapeDtypeStruct((B, 1, H, D), jnp.float32),
        in_specs=[pl.BlockSpec(memory_space=pltpu.VMEM)] * 5,
        out_specs=pl.BlockSpec(memory_space=pltpu.VMEM),
        scratch_shapes=[
            pltpu.VMEM((2, B, H, D), jnp.float32),
            pltpu.VMEM((2, 2, B, H), jnp.float32),
            pltpu.SemaphoreType.DMA((2,)),
            pltpu.SemaphoreType.DMA((2,)),
        ],
        compiler_params=pltpu.CompilerParams(collective_id=0),
    )(Q, K, V, bt, lens2)


# baseline (device time: 58545 ns/iter reference)
import jax
import jax.numpy as jnp
from jax import lax
from jax.experimental import pallas as pl
from jax.experimental.pallas import tpu as pltpu

B, H, D, BS = 16, 16, 64, 16
NB = 128
P_LOCAL = 128
NKEY = P_LOCAL * BS
NEG = -1e30


def kernel(Q, K, V, bt, lens):
    qh = Q[:, 0].transpose(1, 0, 2)
    kh = K.reshape(NKEY, H, D).transpose(1, 2, 0)
    vh = V.reshape(NKEY, H, D).transpose(1, 0, 2)
    bt3 = bt.reshape(B, NB, 1)
    lens3 = lens.reshape(B, 1, 1)

    def body(q_ref, k_ref, v_ref, bt_ref, lens_ref, out_ref,
             acc_comm, stats_comm, send_sems, recv_sems):
        my_x = lax.axis_index("x")
        my_y = lax.axis_index("y")
        my_z = lax.axis_index("z")
        nbr = (1 - my_x, my_y, my_z)

        barrier = pltpu.get_barrier_semaphore()
        pl.semaphore_signal(barrier, inc=1, device_id=nbr,
                            device_id_type=pl.DeviceIdType.MESH)
        pl.semaphore_wait(barrier, 1)

        btv = bt_ref[...]
        lensv = lens_ref[...]
        slot_iota = lax.broadcasted_iota(jnp.int32, (B, NB, P_LOCAL), 1)
        page_iota = lax.broadcasted_iota(jnp.int32, (B, NB, P_LOCAL), 2)
        local_bt = jnp.broadcast_to(btv - my_x * P_LOCAL, (B, NB, P_LOCAL))
        slot_ok = slot_iota < jnp.broadcast_to(lensv, (B, NB, P_LOCAL))
        hit = (local_bt == page_iota) & slot_ok
        counts = jnp.sum(hit.astype(jnp.float32), axis=1)

        row = lax.broadcasted_iota(jnp.int32, (P_LOCAL, NKEY), 0)
        col = lax.broadcasted_iota(jnp.int32, (P_LOCAL, NKEY), 1)
        expand = ((col >= row * BS) & (col < row * BS + BS)).astype(jnp.float32)
        w = jnp.dot(counts, expand,
                    preferred_element_type=jnp.float32)
        sel = w > 0.0

        scale = D ** -0.5
        ms, ls, accs = [], [], []
        for h in range(H):
            s = jnp.dot(q_ref[h], k_ref[h],
                        preferred_element_type=jnp.float32) * scale
            m = jnp.max(jnp.where(sel, s, NEG), axis=-1, keepdims=True)
            e = w * jnp.exp(jnp.where(sel, s - m, NEG))
            l = jnp.sum(e, axis=-1, keepdims=True)
            acc = jnp.dot(e, v_ref[h],
                          preferred_element_type=jnp.float32)
            acc_comm[0, h] = acc
            stats_comm[0, 0, h] = m
            stats_comm[0, 1, h] = l
            ms.append(m)
            ls.append(l)
            accs.append(acc)

        copies = [
            pltpu.make_async_remote_copy(
                src_ref=ref.at[0], dst_ref=ref.at[1],
                send_sem=send_sems.at[i], recv_sem=recv_sems.at[i],
                device_id=nbr, device_id_type=pl.DeviceIdType.MESH,
            )
            for i, ref in enumerate((acc_comm, stats_comm))
        ]
        for c in copies:
            c.start()
        for c in copies:
            c.wait()

        for h in range(H):
            m2 = stats_comm[1, 0, h]
            l2 = stats_comm[1, 1, h]
            acc2 = acc_comm[1, h]
            mm = jnp.maximum(ms[h], m2)
            a1 = jnp.exp(ms[h] - mm)
            a2 = jnp.exp(m2 - mm)
            ll = ls[h] * a1 + l2 * a2
            out_ref[h] = (accs[h] * a1 + acc2 * a2) / ll

    out = pl.pallas_call(
        body,
        out_shape=jax.ShapeDtypeStruct((H, B, D), jnp.float32),
        in_specs=[pl.BlockSpec(memory_space=pltpu.VMEM)] * 5,
        out_specs=pl.BlockSpec(memory_space=pltpu.VMEM),
        scratch_shapes=[
            pltpu.VMEM((2, H, B, D), jnp.float32),
            pltpu.VMEM((2, 2, H, B, 1), jnp.float32),
            pltpu.SemaphoreType.DMA((2,)),
            pltpu.SemaphoreType.DMA((2,)),
        ],
        compiler_params=pltpu.CompilerParams(collective_id=0),
    )(qh, kh, vh, bt3, lens3)
    return out.transpose(1, 0, 2)[:, None]
